# Initial kernel scaffold; baseline (speedup 1.0000x reference)
import jax
import jax.numpy as jnp
from jax.experimental import pallas as pl


def kernel(data, W):
    raise NotImplementedError("write your pallas kernel here")


# speedup vs baseline: 1.0945x; 1.0945x over previous
import functools

import jax
import jax.numpy as jnp
from jax import lax
from jax.experimental import pallas as pl
from jax.experimental.pallas import tpu as pltpu
from jax.experimental.pallas import tpu_sc as plsc

_NC = 2
_NS = 16
_NW = _NC * _NS

_GW = 128
_K = 8


@functools.partial(jax.jit, static_argnums=(2, 3))
def _gather_flat(W, idx, B, D):
    b_per_w = B // _NW
    sup = _K * _GW
    n_sup = b_per_w // sup
    mesh = plsc.VectorSubcoreMesh(
        core_axis_name="c", subcore_axis_name="s",
        num_cores=_NC, num_subcores=_NS,
    )

    @functools.partial(
        pl.kernel,
        out_type=jax.ShapeDtypeStruct((B, D), jnp.float32),
        mesh=mesh,
        scratch_types=[
            pltpu.VMEM((_K, _GW), jnp.int32),
            pltpu.VMEM((sup, D), jnp.float32),
            pltpu.SemaphoreType.DMA,
        ],
        compiler_params=pltpu.CompilerParams(use_tc_tiling_on_sc=False),
    )
    def k(table_hbm, idx_hbm, out_hbm, idx_v, rows_v, sem):
        wid = lax.axis_index("s") * _NC + lax.axis_index("c")
        base = wid * b_per_w

        def body(i, carry):
            off = base + i * sup
            row_off = pl.multiple_of((base // _GW) + i * _K, 8)
            pltpu.sync_copy(idx_hbm.at[pl.ds(row_off, _K)], idx_v)
            copies = [
                pltpu.async_copy(
                    table_hbm.at[idx_v.at[j]],
                    rows_v.at[pl.ds(j * _GW, _GW)],
                    sem,
                )
                for j in range(_K)
            ]
            for c in copies:
                c.wait()
            pltpu.sync_copy(rows_v, out_hbm.at[pl.ds(off, sup)])
            return carry

        lax.fori_loop(0, n_sup, body, 0)

    return k(W, idx.reshape(B // _GW, _GW))


def kernel(data, W):
    B, H = data.shape
    D = W.shape[1]
    idx = data.reshape(-1)
    out = _gather_flat(W, idx, B * H, D)
    return out.reshape(B, H, D)


# speedup vs baseline: 1.1082x; 1.0125x over previous
import functools

import jax
import jax.numpy as jnp
from jax import lax
from jax.experimental import pallas as pl
from jax.experimental.pallas import tpu as pltpu
from jax.experimental.pallas import tpu_sc as plsc

_NC = 2
_NS = 16
_NW = _NC * _NS

_GW = 128
_K = 10


@functools.partial(jax.jit, static_argnums=(2, 3))
def _gather_flat(W, idx, B, D):
    b_per_w = B // _NW
    sup = _K * _GW
    n_sup = b_per_w // sup
    mesh = plsc.VectorSubcoreMesh(
        core_axis_name="c", subcore_axis_name="s",
        num_cores=_NC, num_subcores=_NS,
    )

    @functools.partial(
        pl.kernel,
        out_type=jax.ShapeDtypeStruct((B, D), jnp.float32),
        mesh=mesh,
        scratch_types=[
            pltpu.VMEM((2, _K, _GW), jnp.int32),
            pltpu.VMEM((2, sup, D), jnp.float32),
            pltpu.SemaphoreType.DMA,
        ],
        compiler_params=pltpu.CompilerParams(use_tc_tiling_on_sc=False),
    )
    def k(table_hbm, idx_hbm, out_hbm, idx_v, rows_v, gsem):
        wid = lax.axis_index("s") * _NC + lax.axis_index("c")
        base = wid * b_per_w
        idx_row_base = base // _GW

        def load_idx(c, slot):
            row = pl.multiple_of(idx_row_base + c * _K, 2)
            pltpu.sync_copy(idx_hbm.at[pl.ds(row, _K)], idx_v.at[slot])

        def fire(slot):
            for j in range(_K):
                pltpu.async_copy(
                    table_hbm.at[idx_v.at[slot].at[j]],
                    rows_v.at[slot].at[pl.ds(j * _GW, _GW)],
                    gsem,
                )

        def drain(slot):
            for j in range(_K):
                pltpu.make_async_copy(
                    table_hbm.at[idx_v.at[slot].at[j]],
                    rows_v.at[slot].at[pl.ds(j * _GW, _GW)],
                    gsem,
                ).wait()

        def store_out(c, slot):
            off = base + c * sup
            pltpu.sync_copy(rows_v.at[slot], out_hbm.at[pl.ds(off, sup)])

        load_idx(0, 0)
        fire(0)

        def body(p, carry):
            a = 2 * p
            b = a + 1

            load_idx(b, 1)
            fire(1)

            drain(0)
            store_out(a, 0)

            @pl.when(a + 2 < n_sup)
            def _():
                load_idx(a + 2, 0)
                fire(0)

            drain(1)
            store_out(b, 1)
            return carry

        lax.fori_loop(0, n_sup // 2, body, 0)

    return k(W, idx.reshape(B // _GW, _GW))


def kernel(data, W):
    B, H = data.shape
    D = W.shape[1]
    idx = data.reshape(-1)
    out = _gather_flat(W, idx, B * H, D)
    return out.reshape(B, H, D)


# speedup vs baseline: 1.1084x; 1.0002x over previous
import functools

import jax
import jax.numpy as jnp
from jax import lax
from jax.experimental import pallas as pl
from jax.experimental.pallas import tpu as pltpu
from jax.experimental.pallas import tpu_sc as plsc

_NC = 2
_NS = 16
_NW = _NC * _NS

_GW = 128
_K = 10


@functools.partial(jax.jit, static_argnums=(2, 3))
def _gather_flat(W, idx, B, D):
    b_per_w = B // _NW
    sup = _K * _GW
    n_sup = b_per_w // sup
    mesh = plsc.VectorSubcoreMesh(
        core_axis_name="c", subcore_axis_name="s",
        num_cores=_NC, num_subcores=_NS,
    )

    @functools.partial(
        pl.kernel,
        out_type=jax.ShapeDtypeStruct((B, D), jnp.float32),
        mesh=mesh,
        scratch_types=[
            pltpu.VMEM((2, sup), jnp.int32),
            pltpu.VMEM((2, sup, D), jnp.float32),
            pltpu.SemaphoreType.DMA,
        ],
        compiler_params=pltpu.CompilerParams(use_tc_tiling_on_sc=False),
    )
    def k(table_hbm, idx_hbm, out_hbm, idx_v, rows_v, gsem):
        wid = lax.axis_index("s") * _NC + lax.axis_index("c")
        base = wid * b_per_w

        def load_idx(c, slot):
            off = pl.multiple_of(base + c * sup, 8)
            pltpu.sync_copy(idx_hbm.at[pl.ds(off, sup)], idx_v.at[slot])

        def fire(slot):
            pltpu.async_copy(
                table_hbm.at[idx_v.at[slot]],
                rows_v.at[slot],
                gsem,
            )

        def drain(slot):
            pltpu.make_async_copy(
                table_hbm.at[idx_v.at[slot]],
                rows_v.at[slot],
                gsem,
            ).wait()

        def store_out(c, slot):
            off = base + c * sup
            pltpu.sync_copy(rows_v.at[slot], out_hbm.at[pl.ds(off, sup)])

        load_idx(0, 0)
        fire(0)

        def body(p, carry):
            a = 2 * p
            b = a + 1

            load_idx(b, 1)
            fire(1)

            drain(0)
            store_out(a, 0)

            @pl.when(a + 2 < n_sup)
            def _():
                load_idx(a + 2, 0)
                fire(0)

            drain(1)
            store_out(b, 1)
            return carry

        lax.fori_loop(0, n_sup // 2, body, 0)

    return k(W, idx)


def kernel(data, W):
    B, H = data.shape
    D = W.shape[1]
    idx = data.reshape(-1)
    out = _gather_flat(W, idx, B * H, D)
    return out.reshape(B, H, D)


# speedup vs baseline: 1.7870x; 1.6122x over previous
import functools

import jax
import jax.numpy as jnp
from jax import lax
from jax.experimental import pallas as pl
from jax.experimental.pallas import tpu as pltpu
from jax.experimental.pallas import tpu_sc as plsc

_NC = 2
_NS = 16
_NW = _NC * _NS

_CB = 16


@functools.partial(jax.jit, static_argnums=(2, 3, 4))
def _embed(W, data, B, H, D):
    rows_per_w = B // _NW
    n_sup = rows_per_w // _CB
    mesh = plsc.VectorSubcoreMesh(
        core_axis_name="c", subcore_axis_name="s",
        num_cores=_NC, num_subcores=_NS,
    )

    @functools.partial(
        pl.kernel,
        out_type=jax.ShapeDtypeStruct((B, H, D), jnp.float32),
        mesh=mesh,
        scratch_types=[
            pltpu.VMEM((2, _CB, H), jnp.int32),
            pltpu.VMEM((2, _CB, H, D), jnp.float32),
            pltpu.SemaphoreType.DMA,
        ],
        compiler_params=pltpu.CompilerParams(use_tc_tiling_on_sc=False),
    )
    def k(table_hbm, idx_hbm, out_hbm, idx_v, rows_v, gsem):
        wid = lax.axis_index("s") * _NC + lax.axis_index("c")
        base = wid * rows_per_w

        def load_idx(c, slot):
            row = pl.multiple_of(base + c * _CB, 8)
            pltpu.sync_copy(idx_hbm.at[pl.ds(row, _CB)], idx_v.at[slot])

        def fire(slot):
            for j in range(_CB):
                pltpu.async_copy(
                    table_hbm.at[idx_v.at[slot].at[j]],
                    rows_v.at[slot].at[j],
                    gsem,
                )

        def drain(slot):
            for j in range(_CB):
                pltpu.make_async_copy(
                    table_hbm.at[idx_v.at[slot].at[j]],
                    rows_v.at[slot].at[j],
                    gsem,
                ).wait()

        def store_out(c, slot):
            row = pl.multiple_of(base + c * _CB, 8)
            pltpu.sync_copy(rows_v.at[slot], out_hbm.at[pl.ds(row, _CB)])

        load_idx(0, 0)
        fire(0)

        def body(p, carry):
            a = 2 * p
            b = a + 1

            load_idx(b, 1)
            fire(1)

            drain(0)
            store_out(a, 0)

            @pl.when(a + 2 < n_sup)
            def _():
                load_idx(a + 2, 0)
                fire(0)

            drain(1)
            store_out(b, 1)
            return carry

        lax.fori_loop(0, n_sup // 2, body, 0)

    return k(W, data)


def kernel(data, W):
    B, H = data.shape
    D = W.shape[1]
    return _embed(W, data, B, H, D)


# speedup vs baseline: 1.8014x; 1.0081x over previous
import functools

import jax
import jax.numpy as jnp
from jax import lax
from jax.experimental import pallas as pl
from jax.experimental.pallas import tpu as pltpu
from jax.experimental.pallas import tpu_sc as plsc

_NC = 2
_NS = 16
_NW = _NC * _NS

_CB = 32


@functools.partial(jax.jit, static_argnums=(2, 3, 4))
def _embed(W, data, B, H, D):
    rows_per_w = B // _NW
    n_sup = rows_per_w // _CB
    mesh = plsc.VectorSubcoreMesh(
        core_axis_name="c", subcore_axis_name="s",
        num_cores=_NC, num_subcores=_NS,
    )

    @functools.partial(
        pl.kernel,
        out_type=jax.ShapeDtypeStruct((B, H, D), jnp.float32),
        mesh=mesh,
        scratch_types=[
            pltpu.VMEM((2, _CB, H), jnp.int32),
            pltpu.VMEM((2, _CB, H, D), jnp.float32),
            pltpu.SemaphoreType.DMA,
        ],
        compiler_params=pltpu.CompilerParams(use_tc_tiling_on_sc=False),
    )
    def k(table_hbm, idx_hbm, out_hbm, idx_v, rows_v, gsem):
        wid = lax.axis_index("s") * _NC + lax.axis_index("c")
        base = wid * rows_per_w

        def load_idx(c, slot):
            row = pl.multiple_of(base + c * _CB, 8)
            pltpu.sync_copy(idx_hbm.at[pl.ds(row, _CB)], idx_v.at[slot])

        def fire(slot):
            for j in range(_CB):
                pltpu.async_copy(
                    table_hbm.at[idx_v.at[slot].at[j]],
                    rows_v.at[slot].at[j],
                    gsem,
                )

        def drain(slot):
            for j in range(_CB):
                pltpu.make_async_copy(
                    table_hbm.at[idx_v.at[slot].at[j]],
                    rows_v.at[slot].at[j],
                    gsem,
                ).wait()

        def store_out(c, slot):
            row = pl.multiple_of(base + c * _CB, 8)
            pltpu.sync_copy(rows_v.at[slot], out_hbm.at[pl.ds(row, _CB)])

        load_idx(0, 0)
        fire(0)

        def body(p, carry):
            a = 2 * p
            b = a + 1

            load_idx(b, 1)
            fire(1)

            drain(0)
            store_out(a, 0)

            @pl.when(a + 2 < n_sup)
            def _():
                load_idx(a + 2, 0)
                fire(0)

            drain(1)
            store_out(b, 1)
            return carry

        lax.fori_loop(0, n_sup // 2, body, 0)

    return k(W, data)


def kernel(data, W):
    B, H = data.shape
    D = W.shape[1]
    return _embed(W, data, B, H, D)
